# baseline (speedup 1.0000x reference)
import jax
import jax.numpy as jnp
from jax.experimental import pallas as pl
from jax.experimental.pallas import tpu as pltpu

NODE_DIM = 128
EDGE_DIM = 16
IN_DIM = 272
HIDDEN = 384
DIM_OUT = 128
BLK = 10000


def _mlp_block(xj_ref, xi_ref, ea_ref, w1_ref, b1_ref, w2_ref, b2_ref,
               out_ref):
    xj = xj_ref[:].astype(jnp.bfloat16)
    xi = xi_ref[:].astype(jnp.bfloat16)
    ea = ea_ref[:].astype(jnp.bfloat16)
    x = jnp.concatenate([xj, xi, ea], axis=1)
    h = jnp.dot(x, w1_ref[:].astype(jnp.bfloat16),
                preferred_element_type=jnp.float32)
    h = jnp.maximum((h + b1_ref[:]).astype(jnp.bfloat16), jnp.bfloat16(0.0))
    out_ref[:] = jnp.dot(h, w2_ref[:].astype(jnp.bfloat16),
                         preferred_element_type=jnp.float32) + b2_ref[:]


def kernel(x_i, x_j, edge_attr, edge_index, num_nodes, W1, b1, W2, b2):
    del edge_index, num_nodes
    n_edges = x_i.shape[0]
    grid = (n_edges // BLK,)
    return pl.pallas_call(
        _mlp_block,
        grid=grid,
        in_specs=[
            pl.BlockSpec((BLK, NODE_DIM), lambda i: (i, 0)),
            pl.BlockSpec((BLK, NODE_DIM), lambda i: (i, 0)),
            pl.BlockSpec((BLK, EDGE_DIM), lambda i: (i, 0)),
            pl.BlockSpec((IN_DIM, HIDDEN), lambda i: (0, 0)),
            pl.BlockSpec((1, HIDDEN), lambda i: (0, 0)),
            pl.BlockSpec((HIDDEN, DIM_OUT), lambda i: (0, 0)),
            pl.BlockSpec((1, DIM_OUT), lambda i: (0, 0)),
        ],
        out_specs=pl.BlockSpec((BLK, DIM_OUT), lambda i: (i, 0)),
        out_shape=jax.ShapeDtypeStruct((n_edges, DIM_OUT), jnp.float32),
        compiler_params=pltpu.CompilerParams(
            dimension_semantics=("parallel",),
            vmem_limit_bytes=128 * 1024 * 1024),
    )(x_j, x_i, edge_attr, W1, b1.reshape(1, HIDDEN), W2,
      b2.reshape(1, DIM_OUT))


# speedup vs baseline: 1.0938x; 1.0938x over previous
import jax
import jax.numpy as jnp
from jax.experimental import pallas as pl
from jax.experimental.pallas import tpu as pltpu

NODE_DIM = 128
EDGE_DIM = 16
IN_DIM = 272
HIDDEN = 384
DIM_OUT = 128
BLK = 6400


def _mlp_block(xj_ref, xi_ref, ea_ref, w1_ref, b1_ref, w2_ref, b2_ref,
               out_ref):
    xj = xj_ref[:].astype(jnp.bfloat16)
    xi = xi_ref[:].astype(jnp.bfloat16)
    ea = ea_ref[:].astype(jnp.bfloat16)
    x = jnp.concatenate([xj, xi, ea], axis=1)
    h = jnp.dot(x, w1_ref[:].astype(jnp.bfloat16),
                preferred_element_type=jnp.float32)
    h = jnp.maximum((h + b1_ref[:]).astype(jnp.bfloat16), jnp.bfloat16(0.0))
    out_ref[:] = jnp.dot(h, w2_ref[:].astype(jnp.bfloat16),
                         preferred_element_type=jnp.float32) + b2_ref[:]


def kernel(x_i, x_j, edge_attr, edge_index, num_nodes, W1, b1, W2, b2):
    del edge_index, num_nodes
    n_edges = x_i.shape[0]
    grid = (n_edges // BLK,)
    return pl.pallas_call(
        _mlp_block,
        grid=grid,
        in_specs=[
            pl.BlockSpec((BLK, NODE_DIM), lambda i: (i, 0)),
            pl.BlockSpec((BLK, NODE_DIM), lambda i: (i, 0)),
            pl.BlockSpec((BLK, EDGE_DIM), lambda i: (i, 0)),
            pl.BlockSpec((IN_DIM, HIDDEN), lambda i: (0, 0)),
            pl.BlockSpec((1, HIDDEN), lambda i: (0, 0)),
            pl.BlockSpec((HIDDEN, DIM_OUT), lambda i: (0, 0)),
            pl.BlockSpec((1, DIM_OUT), lambda i: (0, 0)),
        ],
        out_specs=pl.BlockSpec((BLK, DIM_OUT), lambda i: (i, 0)),
        out_shape=jax.ShapeDtypeStruct((n_edges, DIM_OUT), jnp.float32),
        compiler_params=pltpu.CompilerParams(
            dimension_semantics=("parallel",),
            vmem_limit_bytes=128 * 1024 * 1024),
    )(x_j, x_i, edge_attr, W1, b1.reshape(1, HIDDEN), W2,
      b2.reshape(1, DIM_OUT))


# speedup vs baseline: 1.1153x; 1.0196x over previous
import jax
import jax.numpy as jnp
from jax.experimental import pallas as pl
from jax.experimental.pallas import tpu as pltpu

NODE_DIM = 128
EDGE_DIM = 16
IN_DIM = 272
HIDDEN = 384
DIM_OUT = 128
BLK = 8000


def _mlp_block(xj_ref, xi_ref, ea_ref, w1_ref, b1_ref, w2_ref, b2_ref,
               out_ref):
    xj = xj_ref[:].astype(jnp.bfloat16)
    xi = xi_ref[:].astype(jnp.bfloat16)
    ea = ea_ref[:].astype(jnp.bfloat16)
    x = jnp.concatenate([xj, xi, ea], axis=1)
    h = jnp.dot(x, w1_ref[:].astype(jnp.bfloat16),
                preferred_element_type=jnp.float32)
    h = jnp.maximum((h + b1_ref[:]).astype(jnp.bfloat16), jnp.bfloat16(0.0))
    out_ref[:] = jnp.dot(h, w2_ref[:].astype(jnp.bfloat16),
                         preferred_element_type=jnp.float32) + b2_ref[:]


def kernel(x_i, x_j, edge_attr, edge_index, num_nodes, W1, b1, W2, b2):
    del edge_index, num_nodes
    n_edges = x_i.shape[0]
    grid = (n_edges // BLK,)
    return pl.pallas_call(
        _mlp_block,
        grid=grid,
        in_specs=[
            pl.BlockSpec((BLK, NODE_DIM), lambda i: (i, 0)),
            pl.BlockSpec((BLK, NODE_DIM), lambda i: (i, 0)),
            pl.BlockSpec((BLK, EDGE_DIM), lambda i: (i, 0)),
            pl.BlockSpec((IN_DIM, HIDDEN), lambda i: (0, 0)),
            pl.BlockSpec((1, HIDDEN), lambda i: (0, 0)),
            pl.BlockSpec((HIDDEN, DIM_OUT), lambda i: (0, 0)),
            pl.BlockSpec((1, DIM_OUT), lambda i: (0, 0)),
        ],
        out_specs=pl.BlockSpec((BLK, DIM_OUT), lambda i: (i, 0)),
        out_shape=jax.ShapeDtypeStruct((n_edges, DIM_OUT), jnp.float32),
        compiler_params=pltpu.CompilerParams(
            dimension_semantics=("arbitrary",),
            vmem_limit_bytes=128 * 1024 * 1024),
    )(x_j, x_i, edge_attr, W1, b1.reshape(1, HIDDEN), W2,
      b2.reshape(1, DIM_OUT))
